# Initial kernel scaffold; baseline (speedup 1.0000x reference)
import jax
import jax.numpy as jnp
from jax.experimental import pallas as pl


def kernel(user_id, item_id, user_hs, item_hs, beta_u, bias_u, beta_i, bias_i, user_weight, item_weight):
    raise NotImplementedError("write your pallas kernel here")


# speedup vs baseline: 24.1520x; 24.1520x over previous
import functools

import jax
import jax.numpy as jnp
from jax import lax
from jax.experimental import pallas as pl
from jax.experimental.pallas import tpu as pltpu
from jax.experimental.pallas import tpu_sc as plsc

BATCH = 16384
VOCAB = 1000
VPAD = 1024
L = 16
NC, NS = 2, 16
NW = NC * NS
CHUNK = BATCH // NW


def _lr_body(uid_hbm, iid_hbm, hs_u_hbm, hs_i_hbm, bu_hbm, cu_hbm,
             bi_hbm, ci_hbm, wu_hbm, wi_hbm, out_hbm,
             hs_u_v, hs_i_v, bu_v, cu_v, bi_v, ci_v, wu_v, wi_v,
             tab_u_v, tab_i_v, uid_v, iid_v, out_v):
    wid = lax.axis_index("s") * NC + lax.axis_index("c")
    base = wid * CHUNK

    pltpu.sync_copy(uid_hbm.at[pl.ds(base, CHUNK)], uid_v)
    pltpu.sync_copy(iid_hbm.at[pl.ds(base, CHUNK)], iid_v)
    pltpu.sync_copy(hs_u_hbm, hs_u_v)
    pltpu.sync_copy(hs_i_hbm, hs_i_v)
    pltpu.sync_copy(bu_hbm, bu_v)
    pltpu.sync_copy(cu_hbm, cu_v)
    pltpu.sync_copy(bi_hbm, bi_v)
    pltpu.sync_copy(ci_hbm, ci_v)
    pltpu.sync_copy(wu_hbm, wu_v)
    pltpu.sync_copy(wi_hbm, wi_v)

    for j in range(VPAD // L):
        s = pl.ds(j * L, L)
        tab_u_v[s] = wu_v[s] * (bu_v[s] * hs_u_v[s] + cu_v[s])
        tab_i_v[s] = wi_v[s] * (bi_v[s] * hs_i_v[s] + ci_v[s])

    for i in range(CHUNK // L):
        s = pl.ds(i * L, L)
        iu = uid_v[s] - 1
        ii = iid_v[s] - 1
        out_v[s] = plsc.load_gather(tab_u_v, [iu]) + plsc.load_gather(tab_i_v, [ii])

    pltpu.sync_copy(out_v, out_hbm.at[pl.ds(base, CHUNK)])


@functools.partial(
    pl.kernel,
    out_type=jax.ShapeDtypeStruct((BATCH,), jnp.float32),
    mesh=plsc.VectorSubcoreMesh(core_axis_name="c", subcore_axis_name="s"),
    compiler_params=pltpu.CompilerParams(needs_layout_passes=False),
    scratch_types=[pltpu.VMEM((VPAD,), jnp.float32) for _ in range(10)]
    + [pltpu.VMEM((CHUNK,), jnp.int32) for _ in range(2)]
    + [pltpu.VMEM((CHUNK,), jnp.float32)],
)
def _lr_kernel(*refs):
    _lr_body(*refs)


def kernel(user_id, item_id, user_hs, item_hs, beta_u, bias_u,
           beta_i, bias_i, user_weight, item_weight):
    def prep(t):
        return jnp.pad(t.reshape(-1), (0, VPAD - VOCAB))

    out = _lr_kernel(user_id, item_id, prep(user_hs), prep(item_hs),
                     prep(beta_u), prep(bias_u), prep(beta_i), prep(bias_i),
                     prep(user_weight), prep(item_weight))
    return out.reshape(BATCH, 1)


# speedup vs baseline: 29.6664x; 1.2283x over previous
import functools

import jax
import jax.numpy as jnp
from jax import lax
from jax.experimental import pallas as pl
from jax.experimental.pallas import tpu as pltpu
from jax.experimental.pallas import tpu_sc as plsc

BATCH = 16384
VOCAB = 1000
VPAD = 1024
L = 16
NC, NS = 2, 16
NW = NC * NS
CHUNK = BATCH // NW


def _lr_body(uid_hbm, iid_hbm, hs_u_hbm, hs_i_hbm, bu_hbm, cu_hbm,
             bi_hbm, ci_hbm, wu_hbm, wi_hbm, out_hbm,
             hs_u_v, hs_i_v, bu_v, cu_v, bi_v, ci_v, wu_v, wi_v,
             tab_u_v, tab_i_v, uid_v, iid_v, out_v, sem):
    wid = lax.axis_index("s") * NC + lax.axis_index("c")
    base = wid * CHUNK

    copies = [
        pltpu.async_copy(uid_hbm.at[pl.ds(base, CHUNK)], uid_v, sem),
        pltpu.async_copy(iid_hbm.at[pl.ds(base, CHUNK)], iid_v, sem),
        pltpu.async_copy(hs_u_hbm, hs_u_v, sem),
        pltpu.async_copy(hs_i_hbm, hs_i_v, sem),
        pltpu.async_copy(bu_hbm, bu_v, sem),
        pltpu.async_copy(cu_hbm, cu_v, sem),
        pltpu.async_copy(bi_hbm, bi_v, sem),
        pltpu.async_copy(ci_hbm, ci_v, sem),
        pltpu.async_copy(wu_hbm, wu_v, sem),
        pltpu.async_copy(wi_hbm, wi_v, sem),
    ]
    for c in copies:
        c.wait()

    @plsc.parallel_loop(0, VPAD, step=L, unroll=4)
    def _(j):
        s = pl.ds(j, L)
        tab_u_v[s] = wu_v[s] * (bu_v[s] * hs_u_v[s] + cu_v[s])
        tab_i_v[s] = wi_v[s] * (bi_v[s] * hs_i_v[s] + ci_v[s])

    @plsc.parallel_loop(0, CHUNK, step=L, unroll=4)
    def _(i):
        s = pl.ds(i, L)
        out_v[s] = (plsc.load_gather(tab_u_v, [uid_v[s] - 1])
                    + plsc.load_gather(tab_i_v, [iid_v[s] - 1]))

    pltpu.sync_copy(out_v, out_hbm.at[pl.ds(base, CHUNK)])


@functools.partial(
    pl.kernel,
    out_type=jax.ShapeDtypeStruct((BATCH,), jnp.float32),
    mesh=plsc.VectorSubcoreMesh(core_axis_name="c", subcore_axis_name="s"),
    compiler_params=pltpu.CompilerParams(needs_layout_passes=False),
    scratch_types=[pltpu.VMEM((VPAD,), jnp.float32) for _ in range(10)]
    + [pltpu.VMEM((CHUNK,), jnp.int32) for _ in range(2)]
    + [pltpu.VMEM((CHUNK,), jnp.float32), pltpu.SemaphoreType.DMA],
)
def _lr_kernel(*refs):
    _lr_body(*refs)


def kernel(user_id, item_id, user_hs, item_hs, beta_u, bias_u,
           beta_i, bias_i, user_weight, item_weight):
    def prep(t):
        return jnp.pad(t.reshape(-1), (0, VPAD - VOCAB))

    out = _lr_kernel(user_id, item_id, prep(user_hs), prep(item_hs),
                     prep(beta_u), prep(bias_u), prep(beta_i), prep(bias_i),
                     prep(user_weight), prep(item_weight))
    return out.reshape(BATCH, 1)


# speedup vs baseline: 34.1065x; 1.1497x over previous
import functools

import jax
import jax.numpy as jnp
from jax import lax
from jax.experimental import pallas as pl
from jax.experimental.pallas import tpu as pltpu
from jax.experimental.pallas import tpu_sc as plsc

BATCH = 16384
VOCAB = 1000
L = 16
NC, NS = 2, 16
NW = NC * NS
CHUNK = BATCH // NW
VFULL = (VOCAB // L) * L
VTAIL = VOCAB - L


def _lr_body(uid_hbm, iid_hbm, hs_u_hbm, hs_i_hbm, bu_hbm, cu_hbm,
             bi_hbm, ci_hbm, wu_hbm, wi_hbm, out_hbm,
             hs_u_v, hs_i_v, bu_v, cu_v, bi_v, ci_v, wu_v, wi_v,
             tab_u_v, tab_i_v, uid_v, iid_v, out_v, sem):
    wid = lax.axis_index("s") * NC + lax.axis_index("c")
    base = wid * CHUNK

    copies = [
        pltpu.async_copy(uid_hbm.at[pl.ds(base, CHUNK)], uid_v, sem),
        pltpu.async_copy(iid_hbm.at[pl.ds(base, CHUNK)], iid_v, sem),
        pltpu.async_copy(hs_u_hbm, hs_u_v, sem),
        pltpu.async_copy(hs_i_hbm, hs_i_v, sem),
        pltpu.async_copy(bu_hbm, bu_v, sem),
        pltpu.async_copy(cu_hbm, cu_v, sem),
        pltpu.async_copy(bi_hbm, bi_v, sem),
        pltpu.async_copy(ci_hbm, ci_v, sem),
        pltpu.async_copy(wu_hbm, wu_v, sem),
        pltpu.async_copy(wi_hbm, wi_v, sem),
    ]
    for c in copies:
        c.wait()

    def fuse(s):
        tab_u_v[s] = wu_v[s] * (bu_v[s] * hs_u_v[s] + cu_v[s])
        tab_i_v[s] = wi_v[s] * (bi_v[s] * hs_i_v[s] + ci_v[s])

    @plsc.parallel_loop(0, VFULL, step=L, unroll=4)
    def _(j):
        fuse(pl.ds(j, L))

    fuse(pl.ds(VTAIL, L))

    @plsc.parallel_loop(0, CHUNK, step=L, unroll=4)
    def _(i):
        s = pl.ds(i, L)
        out_v[s] = (plsc.load_gather(tab_u_v, [uid_v[s] - 1])
                    + plsc.load_gather(tab_i_v, [iid_v[s] - 1]))

    pltpu.sync_copy(out_v, out_hbm.at[pl.ds(base, CHUNK)])


@functools.partial(
    pl.kernel,
    out_type=jax.ShapeDtypeStruct((BATCH,), jnp.float32),
    mesh=plsc.VectorSubcoreMesh(core_axis_name="c", subcore_axis_name="s"),
    compiler_params=pltpu.CompilerParams(needs_layout_passes=False),
    scratch_types=[pltpu.VMEM((VOCAB,), jnp.float32) for _ in range(10)]
    + [pltpu.VMEM((CHUNK,), jnp.int32) for _ in range(2)]
    + [pltpu.VMEM((CHUNK,), jnp.float32), pltpu.SemaphoreType.DMA],
)
def _lr_kernel(*refs):
    _lr_body(*refs)


def kernel(user_id, item_id, user_hs, item_hs, beta_u, bias_u,
           beta_i, bias_i, user_weight, item_weight):
    out = _lr_kernel(user_id, item_id, user_hs.reshape(-1), item_hs.reshape(-1),
                     beta_u.reshape(-1), bias_u.reshape(-1),
                     beta_i.reshape(-1), bias_i.reshape(-1),
                     user_weight.reshape(-1), item_weight.reshape(-1))
    return out.reshape(BATCH, 1)


# speedup vs baseline: 34.2636x; 1.0046x over previous
import functools

import jax
import jax.numpy as jnp
from jax import lax
from jax.experimental import pallas as pl
from jax.experimental.pallas import tpu as pltpu
from jax.experimental.pallas import tpu_sc as plsc

BATCH = 16384
VOCAB = 1000
L = 16
NC, NS = 2, 16
NW = NC * NS
CHUNK = BATCH // NW
VFULL = (VOCAB // L) * L
VTAIL = VOCAB - L


def _lr_body(uid_hbm, iid_hbm, hs_u_hbm, hs_i_hbm, bu_hbm, cu_hbm,
             bi_hbm, ci_hbm, wu_hbm, wi_hbm, out_hbm,
             hs_u_v, hs_i_v, bu_v, cu_v, bi_v, ci_v, wu_v, wi_v,
             tab_u_v, tab_i_v, uid_v, iid_v, out_v, sem, sem_ids):
    wid = lax.axis_index("s") * NC + lax.axis_index("c")
    base = wid * CHUNK

    tab_copies = [
        pltpu.async_copy(hs_u_hbm, hs_u_v, sem),
        pltpu.async_copy(hs_i_hbm, hs_i_v, sem),
        pltpu.async_copy(bu_hbm, bu_v, sem),
        pltpu.async_copy(cu_hbm, cu_v, sem),
        pltpu.async_copy(bi_hbm, bi_v, sem),
        pltpu.async_copy(ci_hbm, ci_v, sem),
        pltpu.async_copy(wu_hbm, wu_v, sem),
        pltpu.async_copy(wi_hbm, wi_v, sem),
    ]
    id_copies = [
        pltpu.async_copy(uid_hbm.at[pl.ds(base, CHUNK)], uid_v, sem_ids),
        pltpu.async_copy(iid_hbm.at[pl.ds(base, CHUNK)], iid_v, sem_ids),
    ]
    for c in tab_copies:
        c.wait()

    def fuse(s):
        tab_u_v[s] = wu_v[s] * (bu_v[s] * hs_u_v[s] + cu_v[s])
        tab_i_v[s] = wi_v[s] * (bi_v[s] * hs_i_v[s] + ci_v[s])

    @plsc.parallel_loop(0, VFULL, step=L, unroll=4)
    def _(j):
        fuse(pl.ds(j, L))

    fuse(pl.ds(VTAIL, L))

    for c in id_copies:
        c.wait()

    @plsc.parallel_loop(0, CHUNK, step=L, unroll=4)
    def _(i):
        s = pl.ds(i, L)
        out_v[s] = (plsc.load_gather(tab_u_v, [uid_v[s] - 1])
                    + plsc.load_gather(tab_i_v, [iid_v[s] - 1]))

    pltpu.sync_copy(out_v, out_hbm.at[pl.ds(base, CHUNK)])


@functools.partial(
    pl.kernel,
    out_type=jax.ShapeDtypeStruct((BATCH,), jnp.float32),
    mesh=plsc.VectorSubcoreMesh(core_axis_name="c", subcore_axis_name="s"),
    compiler_params=pltpu.CompilerParams(needs_layout_passes=False),
    scratch_types=[pltpu.VMEM((VOCAB,), jnp.float32) for _ in range(10)]
    + [pltpu.VMEM((CHUNK,), jnp.int32) for _ in range(2)]
    + [pltpu.VMEM((CHUNK,), jnp.float32),
       pltpu.SemaphoreType.DMA, pltpu.SemaphoreType.DMA],
)
def _lr_kernel(*refs):
    _lr_body(*refs)


def kernel(user_id, item_id, user_hs, item_hs, beta_u, bias_u,
           beta_i, bias_i, user_weight, item_weight):
    out = _lr_kernel(user_id, item_id, user_hs.reshape(-1), item_hs.reshape(-1),
                     beta_u.reshape(-1), bias_u.reshape(-1),
                     beta_i.reshape(-1), bias_i.reshape(-1),
                     user_weight.reshape(-1), item_weight.reshape(-1))
    return out.reshape(BATCH, 1)


# speedup vs baseline: 34.7717x; 1.0148x over previous
import functools

import jax
import jax.numpy as jnp
from jax import lax
from jax.experimental import pallas as pl
from jax.experimental.pallas import tpu as pltpu
from jax.experimental.pallas import tpu_sc as plsc

BATCH = 16384
VOCAB = 1000
L = 16
NC, NS = 2, 16
NW = NC * NS
CHUNK = BATCH // NW


def _lr_body(uid_hbm, iid_hbm, hs_u_hbm, hs_i_hbm, bu_hbm, cu_hbm,
             bi_hbm, ci_hbm, wu_hbm, wi_hbm, out_hbm,
             hs_u_v, hs_i_v, bu_v, cu_v, bi_v, ci_v, wu_v, wi_v,
             uid_v, iid_v, out_v, sem):
    wid = lax.axis_index("s") * NC + lax.axis_index("c")
    base = wid * CHUNK

    copies = [
        pltpu.async_copy(uid_hbm.at[pl.ds(base, CHUNK)], uid_v, sem),
        pltpu.async_copy(iid_hbm.at[pl.ds(base, CHUNK)], iid_v, sem),
        pltpu.async_copy(hs_u_hbm, hs_u_v, sem),
        pltpu.async_copy(hs_i_hbm, hs_i_v, sem),
        pltpu.async_copy(bu_hbm, bu_v, sem),
        pltpu.async_copy(cu_hbm, cu_v, sem),
        pltpu.async_copy(bi_hbm, bi_v, sem),
        pltpu.async_copy(ci_hbm, ci_v, sem),
        pltpu.async_copy(wu_hbm, wu_v, sem),
        pltpu.async_copy(wi_hbm, wi_v, sem),
    ]
    for c in copies:
        c.wait()

    @plsc.parallel_loop(0, CHUNK, step=L, unroll=4)
    def _(i):
        s = pl.ds(i, L)
        iu = uid_v[s] - 1
        ii = iid_v[s] - 1
        yu = (plsc.load_gather(bu_v, [iu]) * plsc.load_gather(hs_u_v, [iu])
              + plsc.load_gather(cu_v, [iu]))
        yi = (plsc.load_gather(bi_v, [ii]) * plsc.load_gather(hs_i_v, [ii])
              + plsc.load_gather(ci_v, [ii]))
        out_v[s] = (plsc.load_gather(wu_v, [iu]) * yu
                    + plsc.load_gather(wi_v, [ii]) * yi)

    pltpu.sync_copy(out_v, out_hbm.at[pl.ds(base, CHUNK)])


@functools.partial(
    pl.kernel,
    out_type=jax.ShapeDtypeStruct((BATCH,), jnp.float32),
    mesh=plsc.VectorSubcoreMesh(core_axis_name="c", subcore_axis_name="s"),
    compiler_params=pltpu.CompilerParams(needs_layout_passes=False),
    scratch_types=[pltpu.VMEM((VOCAB,), jnp.float32) for _ in range(8)]
    + [pltpu.VMEM((CHUNK,), jnp.int32) for _ in range(2)]
    + [pltpu.VMEM((CHUNK,), jnp.float32), pltpu.SemaphoreType.DMA],
)
def _lr_kernel(*refs):
    _lr_body(*refs)


def kernel(user_id, item_id, user_hs, item_hs, beta_u, bias_u,
           beta_i, bias_i, user_weight, item_weight):
    out = _lr_kernel(user_id, item_id, user_hs.reshape(-1), item_hs.reshape(-1),
                     beta_u.reshape(-1), bias_u.reshape(-1),
                     beta_i.reshape(-1), bias_i.reshape(-1),
                     user_weight.reshape(-1), item_weight.reshape(-1))
    return out.reshape(BATCH, 1)
